# baseline (speedup 1.0000x reference)
import functools

import jax
import jax.numpy as jnp
from jax import lax
from jax.experimental import pallas as pl
from jax.experimental.pallas import tpu as pltpu
from jax.experimental.pallas import tpu_sc as plsc

SEQ = 200
BATCH = 4096
DIM = 64
OUT_DIM = 128

NC = 2
NS = 16
NW = NC * NS
BPW = BATCH // NW
NBUF = 4
LANES = 16


def _pool_body(x_hbm, table_hbm, out_hbm, idx_v, acc_v,
               b0, b1, b2, b3, s0, s1, s2, s3):
    bufs = (b0, b1, b2, b3)
    sems = (s0, s1, s2, s3)
    wid = lax.axis_index("s") * NC + lax.axis_index("c")
    base = wid * BPW

    pltpu.sync_copy(x_hbm.at[:, pl.ds(base, BPW)], idx_v)

    neg = jnp.full((LANES,), -jnp.inf, dtype=jnp.float32)

    def init_row(i, carry):
        for c in range(DIM // LANES):
            acc_v[i, pl.ds(c * LANES, LANES)] = neg
        return carry

    lax.fori_loop(0, BPW, init_row, 0)

    for k in range(NBUF):
        pltpu.make_async_copy(
            table_hbm.at[idx_v.at[k]], bufs[k], sems[k]).start()

    def group(g, carry):
        for k in range(NBUF):
            s_cur = g * NBUF + k
            pltpu.make_async_copy(
                table_hbm.at[idx_v.at[s_cur]], bufs[k], sems[k]).wait()

            def row(i, c2, _buf=bufs[k]):
                for c in range(DIM // LANES):
                    sl = pl.ds(c * LANES, LANES)
                    acc_v[i, sl] = jnp.maximum(acc_v[i, sl], _buf[i, sl])
                return c2

            lax.fori_loop(0, BPW, row, 0)

            s_next = s_cur + NBUF

            @pl.when(s_next < SEQ)
            def _fire(_buf=bufs[k], _sem=sems[k], _s=s_next):
                pltpu.make_async_copy(
                    table_hbm.at[idx_v.at[_s]], _buf, _sem).start()

        return carry

    lax.fori_loop(0, SEQ // NBUF, group, 0)

    pltpu.sync_copy(acc_v, out_hbm.at[pl.ds(base, BPW), :])


@jax.jit
def _pool(x, table):
    mesh = plsc.VectorSubcoreMesh(core_axis_name="c", subcore_axis_name="s")
    return pl.kernel(
        _pool_body,
        out_type=jax.ShapeDtypeStruct((BATCH, DIM), jnp.float32),
        mesh=mesh,
        scratch_types=[
            pltpu.VMEM((SEQ, BPW), jnp.int32),
            pltpu.VMEM((BPW, DIM), jnp.float32),
        ] + [pltpu.VMEM((BPW, DIM), jnp.float32)] * NBUF
          + [pltpu.SemaphoreType.DMA] * NBUF,
        compiler_params=pltpu.CompilerParams(use_tc_tiling_on_sc=False),
    )(x, table)


def _linear_body(p_ref, w_ref, b_ref, o_ref):
    o_ref[...] = lax.dot_general(
        p_ref[...], w_ref[...], (((1,), (1,)), ((), ())),
        preferred_element_type=jnp.float32) + b_ref[...]


@jax.jit
def _linear(pooled, W, b):
    blk = 512
    return pl.pallas_call(
        _linear_body,
        out_shape=jax.ShapeDtypeStruct((BATCH, OUT_DIM), jnp.float32),
        grid=(BATCH // blk,),
        in_specs=[
            pl.BlockSpec((blk, DIM), lambda i: (i, 0)),
            pl.BlockSpec((OUT_DIM, DIM), lambda i: (0, 0)),
            pl.BlockSpec((1, OUT_DIM), lambda i: (0, 0)),
        ],
        out_specs=pl.BlockSpec((blk, OUT_DIM), lambda i: (i, 0)),
    )(pooled, W, b.reshape(1, OUT_DIM))


def kernel(x, table, W, b):
    x = x.astype(jnp.int32)
    pooled = _pool(x, table)
    return _linear(pooled, W, b)
